# Initial kernel scaffold; baseline (speedup 1.0000x reference)
import jax
import jax.numpy as jnp
from jax.experimental import pallas as pl


def kernel(input_coords, num_atoms):
    raise NotImplementedError("write your pallas kernel here")


# speedup vs baseline: 1.0036x; 1.0036x over previous
import functools

import jax
import jax.numpy as jnp
from jax import lax
from jax.experimental import pallas as pl
from jax.experimental.pallas import tpu as pltpu
from jax.experimental.pallas import tpu_sc as plsc

B = 16
C = 12288
CHUNK = 48
NUM_CHUNKS = C // CHUNK


def _body(coords_hbm, na_hbm, out_hbm, row_v, na_v, out_v):
    c = lax.axis_index("c")
    s = lax.axis_index("s")

    @pl.when(s < 8)
    def _():
        b = c * 8 + s
        pltpu.sync_copy(coords_hbm.at[b], row_v)
        pltpu.sync_copy(na_hbm, na_v)

        iota = lax.iota(jnp.int32, 16)
        nvec = na_v[...]
        bvec = jnp.zeros((16,), jnp.int32) + b
        n_vec = nvec.at[bvec].get(mode="promise_in_bounds")
        thr = 3 * n_vec

        def body(i, accs):
            base = i * CHUNK
            out = []
            for j in range(3):
                off = base + j * 16
                v = row_v[pl.ds(off, 16)]
                m = (iota + off) < thr
                out.append(accs[j] + jnp.where(m, v, 0.0))
            return tuple(out)

        zero = jnp.zeros((16,), jnp.float32)
        accs = lax.fori_loop(0, NUM_CHUNKS, body, (zero, zero, zero))

        out_v[...] = zero
        for j in range(3):
            comp = (iota + j * 16) % 3
            plsc.addupdate_scatter(out_v, [comp], accs[j])

        nf = n_vec.astype(jnp.float32)
        out_v[...] = out_v[...] / nf
        pltpu.sync_copy(out_v, out_hbm.at[b])


@jax.jit
def _center(input_coords, num_atoms):
    mesh = plsc.VectorSubcoreMesh(core_axis_name="c", subcore_axis_name="s")
    padded = pl.kernel(
        _body,
        mesh=mesh,
        out_type=jax.ShapeDtypeStruct((B, 16), jnp.float32),
        scratch_types=[
            pltpu.VMEM((C,), jnp.float32),
            pltpu.VMEM((16,), jnp.int32),
            pltpu.VMEM((16,), jnp.float32),
        ],
        compiler_params=pltpu.CompilerParams(needs_layout_passes=False),
    )(input_coords, num_atoms)
    return padded[:, :3]


def kernel(input_coords, num_atoms):
    return _center(input_coords, num_atoms.astype(jnp.int32))
